# baseline (speedup 1.0000x reference)
import functools

import jax
import jax.numpy as jnp
from jax import lax
from jax.experimental import pallas as pl
from jax.experimental.pallas import tpu as pltpu
from jax.experimental.pallas import tpu_sc as plsc

DIM = 32
NUM_WORKERS = 32
CHUNK = 1600
NBUF = 2


def _sc_gather(idx_flat, table):
    n = idx_flat.shape[0]
    per_w = n // NUM_WORKERS
    nchunks = per_w // CHUNK
    mesh = plsc.VectorSubcoreMesh(core_axis_name="c", subcore_axis_name="s")

    @functools.partial(
        pl.kernel,
        mesh=mesh,
        out_type=jax.ShapeDtypeStruct((n, DIM), jnp.float32),
        scratch_types=[
            pltpu.VMEM((NBUF, CHUNK), jnp.int32),
            pltpu.VMEM((NBUF, CHUNK, DIM), jnp.float32),
            [pltpu.SemaphoreType.DMA] * NBUF,
            [pltpu.SemaphoreType.DMA] * NBUF,
        ],
        compiler_params=pltpu.CompilerParams(use_tc_tiling_on_sc=False),
    )
    def k(idx_hbm, table_hbm, out_hbm, idx_v, rows_v, gsems, osems):
        wid = lax.axis_index("s") * 2 + lax.axis_index("c")
        base = wid * per_w

        def start_gather(i):
            b = i % NBUF
            off = base + i * CHUNK
            pltpu.sync_copy(idx_hbm.at[pl.ds(off, CHUNK)], idx_v.at[b])
            pltpu.async_copy(table_hbm.at[idx_v.at[b]], rows_v.at[b], gsems[b])

        start_gather(0)
        for i in range(nchunks):
            b = i % NBUF
            if i + 1 < nchunks:
                nb = (i + 1) % NBUF
                if i + 1 >= NBUF:
                    pltpu.make_async_copy(
                        rows_v.at[nb],
                        out_hbm.at[pl.ds(base + (i + 1 - NBUF) * CHUNK, CHUNK)],
                        osems[nb],
                    ).wait()
                start_gather(i + 1)
            pltpu.make_async_copy(
                table_hbm.at[idx_v.at[b]], rows_v.at[b], gsems[b]
            ).wait()
            pltpu.async_copy(
                rows_v.at[b], out_hbm.at[pl.ds(base + i * CHUNK, CHUNK)], osems[b]
            )
        for i in range(max(0, nchunks - NBUF), nchunks):
            b = i % NBUF
            pltpu.make_async_copy(
                rows_v.at[b], out_hbm.at[pl.ds(base + i * CHUNK, CHUNK)], osems[b]
            ).wait()

    return k(idx_flat, table)


def kernel(idxs, table):
    b, h = idxs.shape
    out = _sc_gather(idxs.reshape(b * h), table)
    return out.reshape(b, h, DIM)


# speedup vs baseline: 1.6178x; 1.6178x over previous
import functools

import jax
import jax.numpy as jnp
from jax import lax
from jax.experimental import pallas as pl
from jax.experimental.pallas import tpu as pltpu
from jax.experimental.pallas import tpu_sc as plsc

DIM = 32
NUM_WORKERS = 32
ROWS = 32
NBUF = 2


def _sc_gather(idxs, table):
    bsz, hist = idxs.shape
    per_w = bsz // NUM_WORKERS
    nchunks = per_w // ROWS
    mesh = plsc.VectorSubcoreMesh(core_axis_name="c", subcore_axis_name="s")

    @functools.partial(
        pl.kernel,
        mesh=mesh,
        out_type=jax.ShapeDtypeStruct((bsz, hist, DIM), jnp.float32),
        scratch_types=[
            pltpu.VMEM((NBUF, ROWS, hist), jnp.int32),
            pltpu.VMEM((NBUF, ROWS, hist, DIM), jnp.float32),
            [pltpu.SemaphoreType.DMA] * NBUF,
            [pltpu.SemaphoreType.DMA] * NBUF,
        ],
        compiler_params=pltpu.CompilerParams(use_tc_tiling_on_sc=False),
    )
    def k(idx_hbm, table_hbm, out_hbm, idx_v, rows_v, gsems, osems):
        wid = lax.axis_index("s") * 2 + lax.axis_index("c")
        base = wid * per_w

        def start_gather(i):
            b = i % NBUF
            pltpu.sync_copy(idx_hbm.at[pl.ds(base + i * ROWS, ROWS)], idx_v.at[b])
            for j in range(ROWS):
                pltpu.async_copy(
                    table_hbm.at[idx_v.at[b, j]], rows_v.at[b, j], gsems[b]
                )

        def wait_gather(i):
            b = i % NBUF
            pltpu.make_async_copy(
                out_hbm.at[pl.ds(base + i * ROWS, ROWS)], rows_v.at[b], gsems[b]
            ).wait()

        start_gather(0)
        for i in range(nchunks):
            b = i % NBUF
            if i + 1 < nchunks:
                nb = (i + 1) % NBUF
                if i + 1 >= NBUF:
                    pltpu.make_async_copy(
                        rows_v.at[nb],
                        out_hbm.at[pl.ds(base + (i + 1 - NBUF) * ROWS, ROWS)],
                        osems[nb],
                    ).wait()
                start_gather(i + 1)
            wait_gather(i)
            pltpu.async_copy(
                rows_v.at[b], out_hbm.at[pl.ds(base + i * ROWS, ROWS)], osems[b]
            )
        for i in range(max(0, nchunks - NBUF), nchunks):
            b = i % NBUF
            pltpu.make_async_copy(
                rows_v.at[b], out_hbm.at[pl.ds(base + i * ROWS, ROWS)], osems[b]
            ).wait()

    return k(idxs, table)


def kernel(idxs, table):
    return _sc_gather(idxs, table)


# speedup vs baseline: 1.7424x; 1.0770x over previous
import functools

import jax
import jax.numpy as jnp
from jax import lax
from jax.experimental import pallas as pl
from jax.experimental.pallas import tpu as pltpu
from jax.experimental.pallas import tpu_sc as plsc

DIM = 32
LANES = 128
NUM_WORKERS = 32
NBUF = 2


def _sc_gather(idxs_t, table):
    hist, bsz = idxs_t.shape
    ntb = bsz // LANES
    per_w = ntb // NUM_WORKERS
    nsteady = hist // NBUF * NBUF
    mesh = plsc.VectorSubcoreMesh(core_axis_name="c", subcore_axis_name="s")

    @functools.partial(
        pl.kernel,
        mesh=mesh,
        out_type=jax.ShapeDtypeStruct((hist, DIM // 8, ntb, 8, LANES),
                                      jnp.float32),
        scratch_types=[
            pltpu.VMEM((hist, LANES), jnp.int32),
            pltpu.VMEM((NBUF, LANES, DIM), jnp.float32),
            pltpu.VMEM((NBUF, DIM, LANES), jnp.float32),
            [pltpu.SemaphoreType.DMA] * NBUF,
            [pltpu.SemaphoreType.DMA] * NBUF,
        ],
        compiler_params=pltpu.CompilerParams(
            use_tc_tiling_on_sc=False, needs_layout_passes=False
        ),
    )
    def k(idx_hbm, table_hbm, out_hbm, idx_v, gbuf, obuf, gsems, osems):
        wid = lax.axis_index("s") * 2 + lax.axis_index("c")
        iota16 = jnp.arange(16, dtype=jnp.int32)
        rows_c = [iota16 + 16 * bl for bl in range(LANES // 16)]
        diag_c = [
            [16 * bc + ((iota16 + k) & 15) for k in range(16)]
            for bc in range(DIM // 16)
        ]

        def transpose_block(b):
            for bl in range(LANES // 16):
                for bc in range(DIM // 16):
                    for kk in range(16):
                        cols = diag_c[bc][kk]
                        v = plsc.load_gather(gbuf.at[b], [rows_c[bl], cols])
                        plsc.store_scatter(obuf.at[b], [cols, rows_c[bl]], v)

        def start_gather(j, b):
            pltpu.async_copy(table_hbm.at[idx_v.at[j]], gbuf.at[b], gsems[b])

        def wait_gather(b):
            pltpu.make_async_copy(
                table_hbm.at[idx_v.at[0]], gbuf.at[b], gsems[b]
            ).wait()

        def start_out(j, b, tb):
            for tc in range(DIM // 8):
                pltpu.async_copy(
                    obuf.at[b, pl.ds(tc * 8, 8)],
                    out_hbm.at[j, tc, tb],
                    osems[b],
                )

        def wait_out(j, b, tb):
            for tc in range(DIM // 8):
                pltpu.make_async_copy(
                    obuf.at[b, pl.ds(tc * 8, 8)],
                    out_hbm.at[j, tc, tb],
                    osems[b],
                ).wait()

        def tbody(t, _):
            tb = t * NUM_WORKERS + wid
            pltpu.sync_copy(idx_hbm.at[:, pl.ds(tb * LANES, LANES)], idx_v)
            for b in range(NBUF):
                start_gather(b, b)

            def jbody(k2, _):
                for b in range(NBUF):
                    j = NBUF * k2 + b
                    wait_gather(b)
                    transpose_block(b)
                    plsc.subcore_barrier()
                    start_out(j, b, tb)
                    start_gather(jnp.minimum(j + NBUF, hist - 1), b)
                for b in range(NBUF):
                    j = NBUF * k2 + b
                    wait_out(j, b, tb)
                return ()

            lax.fori_loop(0, hist // NBUF, jbody, ())
            for b in range(hist - nsteady):
                wait_gather(b)
                transpose_block(b)
                plsc.subcore_barrier()
                start_out(nsteady + b, b, tb)
            for b in range(hist - nsteady):
                wait_out(nsteady + b, b, tb)
            for b in range(hist - nsteady, NBUF):
                wait_gather(b)
            return ()

        lax.fori_loop(0, per_w, tbody, ())

    return k(idxs_t, table)


def kernel(idxs, table):
    bsz, hist = idxs.shape
    out5 = _sc_gather(idxs.T, table)
    return jnp.transpose(out5, (2, 4, 0, 1, 3)).reshape(bsz, hist, DIM)
